# baseline (speedup 1.0000x reference)
import functools

import jax
import jax.numpy as jnp
from jax import lax
from jax.experimental import pallas as pl
from jax.experimental.pallas import tpu as pltpu
from jax.experimental.pallas import tpu_sc as plsc

_N = 10000
_E = 320000
_D = 128

_NC = 2
_NS = 16
_NW = _NC * _NS
_CH = 64
_NCHUNK = 160
_NRING = 4
_EPWP = _CH * _NCHUNK
_NPAD = _EPWP - _E // _NW
_NP = 10240
_RPT = _NP // _NS


def _fill_vmem(ref, rows, cols, value):
    v = jnp.full((16,), value, jnp.float32)

    def row(i, _):
        def col(j, __):
            ref[i, pl.ds(j * 16, 16)] = v
            return 0
        return lax.fori_loop(0, cols // 16, col, 0)

    lax.fori_loop(0, rows, row, 0)


def _zero_acc(zbuf, acc, s):
    for k in range(_RPT // _CH):
        pltpu.sync_copy(zbuf, acc.at[pl.ds(s * _RPT + k * _CH, _CH)])


def _write_out(acc, out_hbm, c, s):
    pltpu.sync_copy(acc.at[pl.ds(s * _RPT, _RPT)],
                    out_hbm.at[c, pl.ds(s * _RPT, _RPT)])


def _cnt_phase(idx_hbm, cnt_hbm, ia, ib, ones_v, acc, isem, ssem, wid, c, s):
    pltpu.sync_copy(idx_hbm.at[wid, 0], ia)
    pltpu.sync_copy(idx_hbm.at[wid, 1], ib)

    def pair(i, _):
        k = i * 2
        pltpu.async_copy(ones_v, acc.at[ia.at[1]], ssem, add=True)
        pltpu.async_copy(ones_v, acc.at[ib.at[1]], ssem, add=True)
        pltpu.make_async_copy(ones_v, acc.at[ia.at[1]], ssem).wait()
        pltpu.async_copy(idx_hbm.at[wid, k + 2], ia, isem)
        pltpu.make_async_copy(ones_v, acc.at[ib.at[1]], ssem).wait()
        pltpu.async_copy(idx_hbm.at[wid, k + 3], ib, isem)
        pltpu.make_async_copy(idx_hbm.at[wid, 0], ia, isem).wait()
        pltpu.make_async_copy(idx_hbm.at[wid, 0], ib, isem).wait()
        return 0

    lax.fori_loop(0, _NCHUNK // 2, pair, 0)
    plsc.subcore_barrier()
    _write_out(acc, cnt_hbm, c, s)


def _agg_phase(x_hbm, idx_hbm, out_hbm, ib_list, rb_list, acc, isem, gsem,
               wid, c, s):
    for b in range(_NRING - 1):
        pltpu.sync_copy(idx_hbm.at[wid, b], ib_list[b])
    for b in range(_NRING - 1):
        pltpu.async_copy(x_hbm.at[ib_list[b].at[0]], rb_list[b], gsem)
    pltpu.async_copy(idx_hbm.at[wid, _NRING - 1], ib_list[_NRING - 1], isem)

    def ring(t, _):
        j = t * _NRING
        for b in range(_NRING):
            cur_i, cur_r = ib_list[b], rb_list[b]
            nxt_i = ib_list[(b + _NRING - 1) % _NRING]
            nxt_r = rb_list[(b + _NRING - 1) % _NRING]
            pltpu.make_async_copy(x_hbm.at[cur_i.at[0]], cur_r, gsem).wait()
            pltpu.make_async_copy(idx_hbm.at[wid, 0], nxt_i, isem).wait()
            pltpu.async_copy(x_hbm.at[nxt_i.at[0]], nxt_r, gsem)
            pltpu.sync_copy(cur_r, acc.at[cur_i.at[1]], add=True)
            pltpu.async_copy(idx_hbm.at[wid, j + b + _NRING], cur_i, isem)
        return 0

    lax.fori_loop(0, _NCHUNK // _NRING, ring, 0)

    for b in range(_NRING - 1):
        pltpu.make_async_copy(x_hbm.at[ib_list[0].at[0]], rb_list[b],
                              gsem).wait()
    pltpu.make_async_copy(idx_hbm.at[wid, 0], ib_list[0], isem).wait()
    plsc.subcore_barrier()
    _write_out(acc, out_hbm, c, s)


def _sc_agg_body(with_cnt, x_hbm, idx_hbm, *rest):
    if with_cnt:
        out_hbm, cnt_hbm, i0, i1, i2, i3, r0, r1, r2, r3, acc, isem, gsem = rest
    else:
        out_hbm, i0, i1, i2, i3, r0, r1, r2, r3, acc, isem, gsem = rest
    ib_list = [i0, i1, i2, i3]
    rb_list = [r0, r1, r2, r3]

    c = lax.axis_index("c")
    s = lax.axis_index("s")
    wid = s * _NC + c

    _fill_vmem(r1, _CH, _D, 0.0)
    _zero_acc(r1, acc, s)
    plsc.subcore_barrier()

    if with_cnt:
        _fill_vmem(r0, _CH, _D, 1.0)
        _cnt_phase(idx_hbm, cnt_hbm, i0, i1, r0, acc, isem, gsem, wid, c, s)
        plsc.subcore_barrier()
        _fill_vmem(r1, _CH, _D, 0.0)
        _zero_acc(r1, acc, s)
        plsc.subcore_barrier()

    _agg_phase(x_hbm, idx_hbm, out_hbm, ib_list, rb_list, acc, isem, gsem,
               wid, c, s)


def _make_sc_agg(with_cnt):
    out_type = [jax.ShapeDtypeStruct((_NC, _NP, _D), jnp.float32)]
    if with_cnt:
        out_type.append(jax.ShapeDtypeStruct((_NC, _NP, _D), jnp.float32))
    scratch = ([pltpu.VMEM((2, _CH), jnp.int32) for _ in range(_NRING)]
               + [pltpu.VMEM((_CH, _D), jnp.float32) for _ in range(_NRING)]
               + [pltpu.VMEM_SHARED((_NP, _D), jnp.float32),
                  pltpu.SemaphoreType.DMA,
                  pltpu.SemaphoreType.DMA])
    return pl.kernel(
        functools.partial(_sc_agg_body, with_cnt),
        out_type=tuple(out_type) if with_cnt else out_type[0],
        mesh=plsc.VectorSubcoreMesh(core_axis_name="c", subcore_axis_name="s"),
        scratch_types=scratch,
    )


_sc_agg_with_cnt = _make_sc_agg(True)
_sc_agg = _make_sc_agg(False)


def _pad_edges(edge_index):
    src = edge_index[0].reshape(_NW, _E // _NW)
    dst = edge_index[1].reshape(_NW, _E // _NW)
    pad_src = (jnp.arange(_NW * _NPAD, dtype=jnp.int32) % _N).reshape(_NW, _NPAD)
    pad_dst = (_N + jnp.arange(_NW * _NPAD, dtype=jnp.int32) % (_NP - _N)
               ).reshape(_NW, _NPAD)
    src = jnp.concatenate([src, pad_src], axis=1).reshape(_NW, _NCHUNK, _CH)
    dst = jnp.concatenate([dst, pad_dst], axis=1).reshape(_NW, _NCHUNK, _CH)
    idx4 = jnp.stack([src, dst], axis=2)
    idx4 = jnp.concatenate([idx4, idx4[:, :_NRING]], axis=1)
    return idx4


_RB = 2000


def _tc_layer1_body(x_ref, p_ref, cnt_ref, wl_ref, bl_ref, wr_ref, h_ref):
    cnt = cnt_ref[0, :, 0:1] + cnt_ref[1, :, 0:1]
    mean = (p_ref[0] + p_ref[1]) / jnp.maximum(cnt, 1.0)
    dn = (((1,), (1,)), ((), ()))
    h = (lax.dot_general(x_ref[...], wl_ref[...], dn,
                         preferred_element_type=jnp.float32)
         + bl_ref[...]
         + lax.dot_general(mean, wr_ref[...], dn,
                           preferred_element_type=jnp.float32))
    h_ref[...] = jnp.maximum(h, 0.0)


def _tc_layer2_body(h_ref, p_ref, cnt_ref, wl_ref, bl_ref, wr_ref,
                    wo_ref, bo_ref, out_ref):
    cnt = cnt_ref[0, :, 0:1] + cnt_ref[1, :, 0:1]
    mean = (p_ref[0] + p_ref[1]) / jnp.maximum(cnt, 1.0)
    dn = (((1,), (1,)), ((), ()))
    h2 = (lax.dot_general(h_ref[...], wl_ref[...], dn,
                          preferred_element_type=jnp.float32)
          + bl_ref[...]
          + lax.dot_general(mean, wr_ref[...], dn,
                            preferred_element_type=jnp.float32))
    h2 = jnp.maximum(h2, 0.0)
    out_ref[...] = lax.dot_general(h2, wo_ref[...], dn,
                                   preferred_element_type=jnp.float32) + bo_ref[...]


def _row_spec():
    return pl.BlockSpec((_RB, _D), lambda i: (i, 0))


def _part_spec():
    return pl.BlockSpec((_NC, _RB, _D), lambda i: (0, i, 0))


def _w_spec():
    return pl.BlockSpec((_D, _D), lambda i: (0, 0))


def _b_spec():
    return pl.BlockSpec((_D,), lambda i: (0,))


def _tc_layer1(x, p, cntp, Wl, bl, Wr):
    return pl.pallas_call(
        _tc_layer1_body,
        grid=(_N // _RB,),
        in_specs=[_row_spec(), _part_spec(), _part_spec(),
                  _w_spec(), _b_spec(), _w_spec()],
        out_specs=_row_spec(),
        out_shape=jax.ShapeDtypeStruct((_N, _D), jnp.float32),
    )(x, p, cntp, Wl, bl, Wr)


def _tc_layer2(h, p, cntp, Wl, bl, Wr, Wo, bo):
    return pl.pallas_call(
        _tc_layer2_body,
        grid=(_N // _RB,),
        in_specs=[_row_spec(), _part_spec(), _part_spec(),
                  _w_spec(), _b_spec(), _w_spec(), _w_spec(), _b_spec()],
        out_specs=_row_spec(),
        out_shape=jax.ShapeDtypeStruct((_N, _D), jnp.float32),
    )(h, p, cntp, Wl, bl, Wr, Wo, bo)


def kernel(x, edge_index, W1l, b1l, W1r, W2l, b2l, W2r, Wout, bout):
    idx4 = _pad_edges(edge_index)
    p1, cntp = _sc_agg_with_cnt(x, idx4)
    h = _tc_layer1(x, p1, cntp, W1l, b1l, W1r)
    p2 = _sc_agg(h, idx4)
    return _tc_layer2(h, p2, cntp, W2l, b2l, W2r, Wout, bout)


# speedup vs baseline: 1.3861x; 1.3861x over previous
import functools

import jax
import jax.numpy as jnp
from jax import lax
from jax.experimental import pallas as pl
from jax.experimental.pallas import tpu as pltpu
from jax.experimental.pallas import tpu_sc as plsc

_N = 10000
_E = 320000
_D = 128

_NC = 2
_NS = 16
_NW = _NC * _NS
_CH = 128
_NCHUNK = 80
_EPWP = _CH * _NCHUNK
_NPAD = _EPWP - _E // _NW
_NP = 10240
_RPT = _NP // _NS


def _fill_vmem(ref, rows, cols, value):
    v = jnp.full((16,), value, jnp.float32)

    def row(i, _):
        def col(j, __):
            ref[i, pl.ds(j * 16, 16)] = v
            return 0
        return lax.fori_loop(0, cols // 16, col, 0)

    lax.fori_loop(0, rows, row, 0)


def _zero_acc(zbuf, acc, s):
    for k in range(_RPT // _CH):
        pltpu.sync_copy(zbuf, acc.at[pl.ds(s * _RPT + k * _CH, _CH)])


def _write_out(acc, out_hbm, c, s):
    pltpu.sync_copy(acc.at[pl.ds(s * _RPT, _RPT)],
                    out_hbm.at[c, pl.ds(s * _RPT, _RPT)])


def _cnt_phase(idx_hbm, cnt_hbm, ia, ib, ones_v, acc, isem, ssem, wid, c, s):
    pltpu.sync_copy(idx_hbm.at[wid, 0], ia)
    pltpu.sync_copy(idx_hbm.at[wid, 1], ib)

    def pair(i, _):
        k = i * 2
        pltpu.async_copy(ones_v, acc.at[ia.at[1]], ssem, add=True)
        pltpu.async_copy(ones_v, acc.at[ib.at[1]], ssem, add=True)
        pltpu.make_async_copy(ones_v, acc.at[ia.at[1]], ssem).wait()
        pltpu.async_copy(idx_hbm.at[wid, k + 2], ia, isem)
        pltpu.make_async_copy(ones_v, acc.at[ib.at[1]], ssem).wait()
        pltpu.async_copy(idx_hbm.at[wid, k + 3], ib, isem)
        pltpu.make_async_copy(idx_hbm.at[wid, 0], ia, isem).wait()
        pltpu.make_async_copy(idx_hbm.at[wid, 0], ib, isem).wait()
        return 0

    lax.fori_loop(0, _NCHUNK // 2, pair, 0)
    plsc.subcore_barrier()
    _write_out(acc, cnt_hbm, c, s)


def _agg_phase(x_hbm, idx_hbm, out_hbm, il, r0, r1, acc, isem, gsem,
               wid, c, s):
    rl = (r0, r1)
    pltpu.sync_copy(idx_hbm.at[wid, 0], il[0])
    pltpu.async_copy(x_hbm.at[il[0].at[0]], r0, gsem)
    pltpu.sync_copy(idx_hbm.at[wid, 1], il[1])
    pltpu.async_copy(idx_hbm.at[wid, 2], il[2], isem)

    def quad(t, _):
        j = t * 4
        for b in range(4):
            pltpu.async_copy(x_hbm.at[il[(b + 1) % 4].at[0]],
                             rl[(b + 1) % 2], gsem)
            pltpu.async_copy(idx_hbm.at[wid, j + b + 3], il[(b + 3) % 4],
                             isem)
            pltpu.make_async_copy(x_hbm.at[il[0].at[0]], rl[b % 2],
                                  gsem).wait()
            pltpu.make_async_copy(idx_hbm.at[wid, 0], il[(b + 2) % 4],
                                  isem).wait()
            pltpu.sync_copy(rl[b % 2], acc.at[il[b % 4].at[1]], add=True)
        return 0

    lax.fori_loop(0, _NCHUNK // 4, quad, 0)

    pltpu.make_async_copy(x_hbm.at[il[0].at[0]], r0, gsem).wait()
    pltpu.make_async_copy(idx_hbm.at[wid, 0], il[0], isem).wait()
    plsc.subcore_barrier()
    _write_out(acc, out_hbm, c, s)


def _sc_agg_body(with_cnt, x_hbm, idx_hbm, *rest):
    if with_cnt:
        out_hbm, cnt_hbm, i0, i1, i2, i3, r0, r1, acc, isem, gsem = rest
    else:
        out_hbm, i0, i1, i2, i3, r0, r1, acc, isem, gsem = rest

    c = lax.axis_index("c")
    s = lax.axis_index("s")
    wid = s * _NC + c

    _fill_vmem(r1, _CH, _D, 0.0)
    _zero_acc(r1, acc, s)
    plsc.subcore_barrier()

    if with_cnt:
        _fill_vmem(r0, _CH, _D, 1.0)
        _cnt_phase(idx_hbm, cnt_hbm, i0, i1, r0, acc, isem, gsem, wid, c, s)
        plsc.subcore_barrier()
        _fill_vmem(r1, _CH, _D, 0.0)
        _zero_acc(r1, acc, s)
        plsc.subcore_barrier()

    _agg_phase(x_hbm, idx_hbm, out_hbm, (i0, i1, i2, i3), r0, r1, acc,
               isem, gsem, wid, c, s)


def _make_sc_agg(with_cnt):
    out_type = [jax.ShapeDtypeStruct((_NC, _NP, _D), jnp.float32)]
    if with_cnt:
        out_type.append(jax.ShapeDtypeStruct((_NC, _NP, _D), jnp.float32))
    scratch = ([pltpu.VMEM((2, _CH), jnp.int32) for _ in range(4)]
               + [pltpu.VMEM((_CH, _D), jnp.float32) for _ in range(2)]
               + [pltpu.VMEM_SHARED((_NP, _D), jnp.float32),
                  pltpu.SemaphoreType.DMA,
                  pltpu.SemaphoreType.DMA])
    return pl.kernel(
        functools.partial(_sc_agg_body, with_cnt),
        out_type=tuple(out_type) if with_cnt else out_type[0],
        mesh=plsc.VectorSubcoreMesh(core_axis_name="c", subcore_axis_name="s"),
        scratch_types=scratch,
    )


_sc_agg_with_cnt = _make_sc_agg(True)
_sc_agg = _make_sc_agg(False)


def _pad_edges(edge_index):
    src = edge_index[0].reshape(_NW, _E // _NW)
    dst = edge_index[1].reshape(_NW, _E // _NW)
    pad_src = (jnp.arange(_NW * _NPAD, dtype=jnp.int32) % _N).reshape(_NW, _NPAD)
    pad_dst = (_N + jnp.arange(_NW * _NPAD, dtype=jnp.int32) % (_NP - _N)
               ).reshape(_NW, _NPAD)
    src = jnp.concatenate([src, pad_src], axis=1).reshape(_NW, _NCHUNK, _CH)
    dst = jnp.concatenate([dst, pad_dst], axis=1).reshape(_NW, _NCHUNK, _CH)
    idx4 = jnp.stack([src, dst], axis=2)
    idx4 = jnp.concatenate([idx4, idx4[:, :4]], axis=1)
    return idx4


_RB = 2000


def _tc_layer1_body(x_ref, p_ref, cnt_ref, wl_ref, bl_ref, wr_ref, h_ref):
    cnt = cnt_ref[0, :, 0:1] + cnt_ref[1, :, 0:1]
    mean = (p_ref[0] + p_ref[1]) / jnp.maximum(cnt, 1.0)
    dn = (((1,), (1,)), ((), ()))
    h = (lax.dot_general(x_ref[...], wl_ref[...], dn,
                         preferred_element_type=jnp.float32)
         + bl_ref[...]
         + lax.dot_general(mean, wr_ref[...], dn,
                           preferred_element_type=jnp.float32))
    h_ref[...] = jnp.maximum(h, 0.0)


def _tc_layer2_body(h_ref, p_ref, cnt_ref, wl_ref, bl_ref, wr_ref,
                    wo_ref, bo_ref, out_ref):
    cnt = cnt_ref[0, :, 0:1] + cnt_ref[1, :, 0:1]
    mean = (p_ref[0] + p_ref[1]) / jnp.maximum(cnt, 1.0)
    dn = (((1,), (1,)), ((), ()))
    h2 = (lax.dot_general(h_ref[...], wl_ref[...], dn,
                          preferred_element_type=jnp.float32)
          + bl_ref[...]
          + lax.dot_general(mean, wr_ref[...], dn,
                            preferred_element_type=jnp.float32))
    h2 = jnp.maximum(h2, 0.0)
    out_ref[...] = lax.dot_general(h2, wo_ref[...], dn,
                                   preferred_element_type=jnp.float32) + bo_ref[...]


def _row_spec():
    return pl.BlockSpec((_RB, _D), lambda i: (i, 0))


def _part_spec():
    return pl.BlockSpec((_NC, _RB, _D), lambda i: (0, i, 0))


def _w_spec():
    return pl.BlockSpec((_D, _D), lambda i: (0, 0))


def _b_spec():
    return pl.BlockSpec((_D,), lambda i: (0,))


def _tc_layer1(x, p, cntp, Wl, bl, Wr):
    return pl.pallas_call(
        _tc_layer1_body,
        grid=(_N // _RB,),
        in_specs=[_row_spec(), _part_spec(), _part_spec(),
                  _w_spec(), _b_spec(), _w_spec()],
        out_specs=_row_spec(),
        out_shape=jax.ShapeDtypeStruct((_N, _D), jnp.float32),
    )(x, p, cntp, Wl, bl, Wr)


def _tc_layer2(h, p, cntp, Wl, bl, Wr, Wo, bo):
    return pl.pallas_call(
        _tc_layer2_body,
        grid=(_N // _RB,),
        in_specs=[_row_spec(), _part_spec(), _part_spec(),
                  _w_spec(), _b_spec(), _w_spec(), _w_spec(), _b_spec()],
        out_specs=_row_spec(),
        out_shape=jax.ShapeDtypeStruct((_N, _D), jnp.float32),
    )(h, p, cntp, Wl, bl, Wr, Wo, bo)


def kernel(x, edge_index, W1l, b1l, W1r, W2l, b2l, W2r, Wout, bout):
    idx4 = _pad_edges(edge_index)
    p1, cntp = _sc_agg_with_cnt(x, idx4)
    h = _tc_layer1(x, p1, cntp, W1l, b1l, W1r)
    p2 = _sc_agg(h, idx4)
    return _tc_layer2(h, p2, cntp, W2l, b2l, W2r, Wout, bout)


# speedup vs baseline: 1.6966x; 1.2240x over previous
import functools

import jax
import jax.numpy as jnp
from jax import lax
from jax.experimental import pallas as pl
from jax.experimental.pallas import tpu as pltpu
from jax.experimental.pallas import tpu_sc as plsc

_N = 10000
_E = 320000
_D = 128

_NC = 2
_NS = 16
_NW = _NC * _NS
_CH = 128
_NCHUNK = 80
_EPWP = _CH * _NCHUNK
_NPAD = _EPWP - _E // _NW
_NP = 10240
_RPT = _NP // _NS


def _fill_vmem1d(ref, n, value):
    v = jnp.full((16,), value, jnp.float32)

    def step(i, _):
        ref[pl.ds(i * 16, 16)] = v
        return 0

    lax.fori_loop(0, n // 16, step, 0)


def _fill_vmem(ref, rows, cols, value):
    v = jnp.full((16,), value, jnp.float32)

    def row(i, _):
        def col(j, __):
            ref[i, pl.ds(j * 16, 16)] = v
            return 0
        return lax.fori_loop(0, cols // 16, col, 0)

    lax.fori_loop(0, rows, row, 0)


def _zero_acc(zbuf, acc, s):
    for k in range(_RPT // _CH):
        pltpu.sync_copy(zbuf, acc.at[pl.ds(s * _RPT + k * _CH, _CH)])


def _write_out(acc, out_hbm, c, s):
    pltpu.sync_copy(acc.at[pl.ds(s * _RPT, _RPT)],
                    out_hbm.at[c, pl.ds(s * _RPT, _RPT)])


def _agg_phase(x_hbm, idx_hbm, out_hbm, il, r0, r1, acc, isem, gsem,
               wid, c, s):
    rl = (r0, r1)
    pltpu.sync_copy(idx_hbm.at[wid, 0], il[0])
    pltpu.async_copy(x_hbm.at[il[0].at[0]], r0, gsem)
    pltpu.sync_copy(idx_hbm.at[wid, 1], il[1])
    pltpu.async_copy(idx_hbm.at[wid, 2], il[2], isem)

    def quad(t, _):
        j = t * 4
        for b in range(4):
            pltpu.async_copy(x_hbm.at[il[(b + 1) % 4].at[0]],
                             rl[(b + 1) % 2], gsem)
            pltpu.async_copy(idx_hbm.at[wid, j + b + 3], il[(b + 3) % 4],
                             isem)
            pltpu.make_async_copy(x_hbm.at[il[0].at[0]], rl[b % 2],
                                  gsem).wait()
            pltpu.make_async_copy(idx_hbm.at[wid, 0], il[(b + 2) % 4],
                                  isem).wait()
            pltpu.sync_copy(rl[b % 2], acc.at[il[b % 4].at[1]], add=True)
        return 0

    lax.fori_loop(0, _NCHUNK // 4, quad, 0)

    pltpu.make_async_copy(x_hbm.at[il[0].at[0]], r0, gsem).wait()
    pltpu.make_async_copy(idx_hbm.at[wid, 0], il[0], isem).wait()
    plsc.subcore_barrier()
    _write_out(acc, out_hbm, c, s)


def _sc_agg_body(x_hbm, idx_hbm, out_hbm, i0, i1, i2, i3, r0, r1, acc,
                 isem, gsem):
    c = lax.axis_index("c")
    s = lax.axis_index("s")
    wid = s * _NC + c

    _fill_vmem(r1, _CH, _D, 0.0)
    _zero_acc(r1, acc, s)
    plsc.subcore_barrier()

    _agg_phase(x_hbm, idx_hbm, out_hbm, (i0, i1, i2, i3), r0, r1, acc,
               isem, gsem, wid, c, s)


_sc_agg = pl.kernel(
    _sc_agg_body,
    out_type=jax.ShapeDtypeStruct((_NC, _NP, _D), jnp.float32),
    mesh=plsc.VectorSubcoreMesh(core_axis_name="c", subcore_axis_name="s"),
    scratch_types=(
        [pltpu.VMEM((2, _CH), jnp.int32) for _ in range(4)]
        + [pltpu.VMEM((_CH, _D), jnp.float32) for _ in range(2)]
        + [pltpu.VMEM_SHARED((_NP, _D), jnp.float32),
           pltpu.SemaphoreType.DMA,
           pltpu.SemaphoreType.DMA]
    ),
)


def _sc_cnt_body(dst_hbm, out_hbm, hist, dst_all, tmp, cbuf, shared):
    c = lax.axis_index("c")
    s = lax.axis_index("s")
    wid = s * _NC + c

    _fill_vmem1d(hist, _NP, 0.0)
    pltpu.sync_copy(dst_hbm.at[wid], dst_all)

    def hrow(j, _):
        def hcol(l, __):
            v = dst_all[j, pl.ds(l * 16, 16)]
            cc, last = plsc.scan_count(v)
            plsc.addupdate_scatter(hist, [v], cc.astype(jnp.float32),
                                   mask=last)
            return 0
        return lax.fori_loop(0, _CH // 16, hcol, 0)

    lax.fori_loop(0, _NCHUNK, hrow, 0)

    pltpu.sync_copy(hist, shared.at[s])
    plsc.subcore_barrier()

    _fill_vmem1d(cbuf, _RPT, 0.0)
    for t in range(_NS):
        pltpu.sync_copy(shared.at[t, pl.ds(s * _RPT, _RPT)], tmp)

        def add(v, _):
            cbuf[pl.ds(v * 16, 16)] = cbuf[pl.ds(v * 16, 16)] + tmp[pl.ds(v * 16, 16)]
            return 0
        lax.fori_loop(0, _RPT // 16, add, 0)

    pltpu.sync_copy(cbuf, out_hbm.at[c, s])


_sc_cnt = pl.kernel(
    _sc_cnt_body,
    out_type=jax.ShapeDtypeStruct((_NC, _NS, _RPT), jnp.float32),
    mesh=plsc.VectorSubcoreMesh(core_axis_name="c", subcore_axis_name="s"),
    scratch_types=[
        pltpu.VMEM((_NP,), jnp.float32),
        pltpu.VMEM((_NCHUNK, _CH), jnp.int32),
        pltpu.VMEM((_RPT,), jnp.float32),
        pltpu.VMEM((_RPT,), jnp.float32),
        pltpu.VMEM_SHARED((_NS, _NP), jnp.float32),
    ],
    compiler_params=pltpu.CompilerParams(needs_layout_passes=False),
)


def _pad_edges(edge_index):
    src = edge_index[0].reshape(_NW, _E // _NW)
    dst = edge_index[1].reshape(_NW, _E // _NW)
    pad_src = (jnp.arange(_NW * _NPAD, dtype=jnp.int32) % _N).reshape(_NW, _NPAD)
    pad_dst = (_N + jnp.arange(_NW * _NPAD, dtype=jnp.int32) % (_NP - _N)
               ).reshape(_NW, _NPAD)
    src = jnp.concatenate([src, pad_src], axis=1).reshape(_NW, _NCHUNK, _CH)
    dst = jnp.concatenate([dst, pad_dst], axis=1).reshape(_NW, _NCHUNK, _CH)
    idx4 = jnp.stack([src, dst], axis=2)
    idx4 = jnp.concatenate([idx4, idx4[:, :4]], axis=1)
    return idx4, dst


_RB = 2000


def _tc_layer1_body(x_ref, p_ref, cnt_ref, wl_ref, bl_ref, wr_ref, h_ref):
    cnt = cnt_ref[0] + cnt_ref[1]
    mean = (p_ref[0] + p_ref[1]) / jnp.maximum(cnt, 1.0)
    dn = (((1,), (1,)), ((), ()))
    h = (lax.dot_general(x_ref[...], wl_ref[...], dn,
                         preferred_element_type=jnp.float32)
         + bl_ref[...]
         + lax.dot_general(mean, wr_ref[...], dn,
                           preferred_element_type=jnp.float32))
    h_ref[...] = jnp.maximum(h, 0.0)


def _tc_layer2_body(h_ref, p_ref, cnt_ref, wl_ref, bl_ref, wr_ref,
                    wo_ref, bo_ref, out_ref):
    cnt = cnt_ref[0] + cnt_ref[1]
    mean = (p_ref[0] + p_ref[1]) / jnp.maximum(cnt, 1.0)
    dn = (((1,), (1,)), ((), ()))
    h2 = (lax.dot_general(h_ref[...], wl_ref[...], dn,
                          preferred_element_type=jnp.float32)
          + bl_ref[...]
          + lax.dot_general(mean, wr_ref[...], dn,
                            preferred_element_type=jnp.float32))
    h2 = jnp.maximum(h2, 0.0)
    out_ref[...] = lax.dot_general(h2, wo_ref[...], dn,
                                   preferred_element_type=jnp.float32) + bo_ref[...]


def _row_spec():
    return pl.BlockSpec((_RB, _D), lambda i: (i, 0))


def _part_spec():
    return pl.BlockSpec((_NC, _RB, _D), lambda i: (0, i, 0))


def _cnt_spec():
    return pl.BlockSpec((_NC, _RB, 1), lambda i: (0, i, 0))


def _w_spec():
    return pl.BlockSpec((_D, _D), lambda i: (0, 0))


def _b_spec():
    return pl.BlockSpec((_D,), lambda i: (0,))


def _tc_layer1(x, p, cntp, Wl, bl, Wr):
    return pl.pallas_call(
        _tc_layer1_body,
        grid=(_N // _RB,),
        in_specs=[_row_spec(), _part_spec(), _cnt_spec(),
                  _w_spec(), _b_spec(), _w_spec()],
        out_specs=_row_spec(),
        out_shape=jax.ShapeDtypeStruct((_N, _D), jnp.float32),
    )(x, p, cntp, Wl, bl, Wr)


def _tc_layer2(h, p, cntp, Wl, bl, Wr, Wo, bo):
    return pl.pallas_call(
        _tc_layer2_body,
        grid=(_N // _RB,),
        in_specs=[_row_spec(), _part_spec(), _cnt_spec(),
                  _w_spec(), _b_spec(), _w_spec(), _w_spec(), _b_spec()],
        out_specs=_row_spec(),
        out_shape=jax.ShapeDtypeStruct((_N, _D), jnp.float32),
    )(h, p, cntp, Wl, bl, Wr, Wo, bo)


def kernel(x, edge_index, W1l, b1l, W1r, W2l, b2l, W2r, Wout, bout):
    idx4, dst3 = _pad_edges(edge_index)
    cntp = _sc_cnt(dst3).reshape(_NC, _NP, 1)
    p1 = _sc_agg(x, idx4)
    h = _tc_layer1(x, p1, cntp, W1l, b1l, W1r)
    p2 = _sc_agg(h, idx4)
    return _tc_layer2(h, p2, cntp, W2l, b2l, W2r, Wout, bout)


# speedup vs baseline: 1.6977x; 1.0007x over previous
import functools

import jax
import jax.numpy as jnp
from jax import lax
from jax.experimental import pallas as pl
from jax.experimental.pallas import tpu as pltpu
from jax.experimental.pallas import tpu_sc as plsc

_N = 10000
_E = 320000
_D = 128

_NC = 2
_NS = 16
_NW = _NC * _NS
_CH = 128
_NCHUNK = 80
_EPWP = _CH * _NCHUNK
_NPAD = _EPWP - _E // _NW
_NP = 10240
_RPT = _NP // _NS


def _fill_vmem1d(ref, n, value):
    v = jnp.full((16,), value, jnp.float32)

    def step(i, _):
        ref[pl.ds(i * 16, 16)] = v
        return 0

    lax.fori_loop(0, n // 16, step, 0)


def _fill_vmem(ref, rows, cols, value):
    v = jnp.full((16,), value, jnp.float32)

    def row(i, _):
        def col(j, __):
            ref[i, pl.ds(j * 16, 16)] = v
            return 0
        return lax.fori_loop(0, cols // 16, col, 0)

    lax.fori_loop(0, rows, row, 0)


def _zero_acc(zbuf, acc, s):
    for k in range(_RPT // _CH):
        pltpu.sync_copy(zbuf, acc.at[pl.ds(s * _RPT + k * _CH, _CH)])


def _write_out(acc, out_hbm, c, s):
    pltpu.sync_copy(acc.at[pl.ds(s * _RPT, _RPT)],
                    out_hbm.at[c, pl.ds(s * _RPT, _RPT)])


def _agg_phase(x_hbm, idx_hbm, out_hbm, il, r0, r1, acc, isem, gsem, ssem,
               wid, c, s):
    rl = (r0, r1)
    pltpu.sync_copy(idx_hbm.at[wid, 0], il[0])
    pltpu.async_copy(x_hbm.at[il[0].at[0]], r0, gsem)
    pltpu.sync_copy(idx_hbm.at[wid, 1], il[1])
    pltpu.async_copy(idx_hbm.at[wid, 2], il[2], isem)
    pltpu.async_copy(r1, acc.at[il[0].at[1]], ssem, add=True)

    def quad(t, _):
        j = t * 4
        for b in range(4):
            pltpu.make_async_copy(rl[(b + 1) % 2], acc.at[il[0].at[1]],
                                  ssem).wait()
            pltpu.async_copy(x_hbm.at[il[(b + 1) % 4].at[0]],
                             rl[(b + 1) % 2], gsem)
            pltpu.async_copy(idx_hbm.at[wid, j + b + 3], il[(b + 3) % 4],
                             isem)
            pltpu.make_async_copy(x_hbm.at[il[0].at[0]], rl[b % 2],
                                  gsem).wait()
            pltpu.make_async_copy(idx_hbm.at[wid, 0], il[(b + 2) % 4],
                                  isem).wait()
            pltpu.async_copy(rl[b % 2], acc.at[il[b % 4].at[1]], ssem,
                             add=True)
        return 0

    lax.fori_loop(0, _NCHUNK // 4, quad, 0)

    pltpu.make_async_copy(x_hbm.at[il[0].at[0]], r0, gsem).wait()
    pltpu.make_async_copy(idx_hbm.at[wid, 0], il[0], isem).wait()
    pltpu.make_async_copy(r0, acc.at[il[0].at[1]], ssem).wait()
    plsc.subcore_barrier()
    _write_out(acc, out_hbm, c, s)


def _sc_agg_body(x_hbm, idx_hbm, out_hbm, i0, i1, i2, i3, r0, r1, acc,
                 isem, gsem, ssem):
    c = lax.axis_index("c")
    s = lax.axis_index("s")
    wid = s * _NC + c

    _fill_vmem(r1, _CH, _D, 0.0)
    _zero_acc(r1, acc, s)
    plsc.subcore_barrier()

    _agg_phase(x_hbm, idx_hbm, out_hbm, (i0, i1, i2, i3), r0, r1, acc,
               isem, gsem, ssem, wid, c, s)


_sc_agg = pl.kernel(
    _sc_agg_body,
    out_type=jax.ShapeDtypeStruct((_NC, _NP, _D), jnp.float32),
    mesh=plsc.VectorSubcoreMesh(core_axis_name="c", subcore_axis_name="s"),
    scratch_types=(
        [pltpu.VMEM((2, _CH), jnp.int32) for _ in range(4)]
        + [pltpu.VMEM((_CH, _D), jnp.float32) for _ in range(2)]
        + [pltpu.VMEM_SHARED((_NP, _D), jnp.float32),
           pltpu.SemaphoreType.DMA,
           pltpu.SemaphoreType.DMA,
           pltpu.SemaphoreType.DMA]
    ),
)


def _sc_cnt_body(dst_hbm, out_hbm, hist, dst_all, tmp, cbuf, shared):
    c = lax.axis_index("c")
    s = lax.axis_index("s")
    wid = s * _NC + c

    _fill_vmem1d(hist, _NP, 0.0)
    pltpu.sync_copy(dst_hbm.at[wid], dst_all)

    def hrow(j, _):
        def hcol(l, __):
            v = dst_all[j, pl.ds(l * 16, 16)]
            cc, last = plsc.scan_count(v)
            plsc.addupdate_scatter(hist, [v], cc.astype(jnp.float32),
                                   mask=last)
            return 0
        return lax.fori_loop(0, _CH // 16, hcol, 0)

    lax.fori_loop(0, _NCHUNK, hrow, 0)

    pltpu.sync_copy(hist, shared.at[s])
    plsc.subcore_barrier()

    _fill_vmem1d(cbuf, _RPT, 0.0)
    for t in range(_NS):
        pltpu.sync_copy(shared.at[t, pl.ds(s * _RPT, _RPT)], tmp)

        def add(v, _):
            cbuf[pl.ds(v * 16, 16)] = cbuf[pl.ds(v * 16, 16)] + tmp[pl.ds(v * 16, 16)]
            return 0
        lax.fori_loop(0, _RPT // 16, add, 0)

    pltpu.sync_copy(cbuf, out_hbm.at[c, s])


_sc_cnt = pl.kernel(
    _sc_cnt_body,
    out_type=jax.ShapeDtypeStruct((_NC, _NS, _RPT), jnp.float32),
    mesh=plsc.VectorSubcoreMesh(core_axis_name="c", subcore_axis_name="s"),
    scratch_types=[
        pltpu.VMEM((_NP,), jnp.float32),
        pltpu.VMEM((_NCHUNK, _CH), jnp.int32),
        pltpu.VMEM((_RPT,), jnp.float32),
        pltpu.VMEM((_RPT,), jnp.float32),
        pltpu.VMEM_SHARED((_NS, _NP), jnp.float32),
    ],
    compiler_params=pltpu.CompilerParams(needs_layout_passes=False),
)


def _pad_edges(edge_index):
    src = edge_index[0].reshape(_NW, _E // _NW)
    dst = edge_index[1].reshape(_NW, _E // _NW)
    pad_src = (jnp.arange(_NW * _NPAD, dtype=jnp.int32) % _N).reshape(_NW, _NPAD)
    pad_dst = (_N + jnp.arange(_NW * _NPAD, dtype=jnp.int32) % (_NP - _N)
               ).reshape(_NW, _NPAD)
    src = jnp.concatenate([src, pad_src], axis=1).reshape(_NW, _NCHUNK, _CH)
    dst = jnp.concatenate([dst, pad_dst], axis=1).reshape(_NW, _NCHUNK, _CH)
    idx4 = jnp.stack([src, dst], axis=2)
    idx4 = jnp.concatenate([idx4, idx4[:, :4]], axis=1)
    return idx4, dst


_RB = 2000


def _tc_layer1_body(x_ref, p_ref, cnt_ref, wl_ref, bl_ref, wr_ref, h_ref):
    cnt = cnt_ref[0] + cnt_ref[1]
    mean = (p_ref[0] + p_ref[1]) / jnp.maximum(cnt, 1.0)
    dn = (((1,), (1,)), ((), ()))
    h = (lax.dot_general(x_ref[...], wl_ref[...], dn,
                         preferred_element_type=jnp.float32)
         + bl_ref[...]
         + lax.dot_general(mean, wr_ref[...], dn,
                           preferred_element_type=jnp.float32))
    h_ref[...] = jnp.maximum(h, 0.0)


def _tc_layer2_body(h_ref, p_ref, cnt_ref, wl_ref, bl_ref, wr_ref,
                    wo_ref, bo_ref, out_ref):
    cnt = cnt_ref[0] + cnt_ref[1]
    mean = (p_ref[0] + p_ref[1]) / jnp.maximum(cnt, 1.0)
    dn = (((1,), (1,)), ((), ()))
    h2 = (lax.dot_general(h_ref[...], wl_ref[...], dn,
                          preferred_element_type=jnp.float32)
          + bl_ref[...]
          + lax.dot_general(mean, wr_ref[...], dn,
                            preferred_element_type=jnp.float32))
    h2 = jnp.maximum(h2, 0.0)
    out_ref[...] = lax.dot_general(h2, wo_ref[...], dn,
                                   preferred_element_type=jnp.float32) + bo_ref[...]


def _row_spec():
    return pl.BlockSpec((_RB, _D), lambda i: (i, 0))


def _part_spec():
    return pl.BlockSpec((_NC, _RB, _D), lambda i: (0, i, 0))


def _cnt_spec():
    return pl.BlockSpec((_NC, _RB, 1), lambda i: (0, i, 0))


def _w_spec():
    return pl.BlockSpec((_D, _D), lambda i: (0, 0))


def _b_spec():
    return pl.BlockSpec((_D,), lambda i: (0,))


def _tc_layer1(x, p, cntp, Wl, bl, Wr):
    return pl.pallas_call(
        _tc_layer1_body,
        grid=(_N // _RB,),
        in_specs=[_row_spec(), _part_spec(), _cnt_spec(),
                  _w_spec(), _b_spec(), _w_spec()],
        out_specs=_row_spec(),
        out_shape=jax.ShapeDtypeStruct((_N, _D), jnp.float32),
    )(x, p, cntp, Wl, bl, Wr)


def _tc_layer2(h, p, cntp, Wl, bl, Wr, Wo, bo):
    return pl.pallas_call(
        _tc_layer2_body,
        grid=(_N // _RB,),
        in_specs=[_row_spec(), _part_spec(), _cnt_spec(),
                  _w_spec(), _b_spec(), _w_spec(), _w_spec(), _b_spec()],
        out_specs=_row_spec(),
        out_shape=jax.ShapeDtypeStruct((_N, _D), jnp.float32),
    )(h, p, cntp, Wl, bl, Wr, Wo, bo)


def kernel(x, edge_index, W1l, b1l, W1r, W2l, b2l, W2r, Wout, bout):
    idx4, dst3 = _pad_edges(edge_index)
    cntp = _sc_cnt(dst3).reshape(_NC, _NP, 1)
    p1 = _sc_agg(x, idx4)
    h = _tc_layer1(x, p1, cntp, W1l, b1l, W1r)
    p2 = _sc_agg(h, idx4)
    return _tc_layer2(h, p2, cntp, W2l, b2l, W2r, Wout, bout)


# speedup vs baseline: 1.6980x; 1.0002x over previous
import jax
import jax.numpy as jnp
from jax import lax
from jax.experimental import pallas as pl
from jax.experimental.pallas import tpu as pltpu
from jax.experimental.pallas import tpu_sc as plsc

_N = 10000
_E = 320000
_D = 128

_NC = 2
_NS = 16
_NW = _NC * _NS
_CH = 128
_NCHUNK = 80
_EPWP = _CH * _NCHUNK
_NPAD = _EPWP - _E // _NW
_NP = 10240
_RPT = _NP // _NS


def _fill_vmem1d(ref, n, value):
    v = jnp.full((16,), value, jnp.float32)

    def step(i, _):
        ref[pl.ds(i * 16, 16)] = v
        return 0

    lax.fori_loop(0, n // 16, step, 0)


def _fill_vmem(ref, rows, cols, value):
    v = jnp.full((16,), value, jnp.float32)

    def row(i, _):
        def col(j, __):
            ref[i, pl.ds(j * 16, 16)] = v
            return 0
        return lax.fori_loop(0, cols // 16, col, 0)

    lax.fori_loop(0, rows, row, 0)


def _zero_acc(zbuf, acc, s):
    for k in range(_RPT // _CH):
        pltpu.sync_copy(zbuf, acc.at[pl.ds(s * _RPT + k * _CH, _CH)])


def _write_out(acc, out_hbm, c, s):
    pltpu.sync_copy(acc.at[pl.ds(s * _RPT, _RPT)],
                    out_hbm.at[c, pl.ds(s * _RPT, _RPT)])


def _agg_phase(x_hbm, idx_hbm, out_hbm, il, r0, r1, acc, isem, gsem, ssem,
               wid, c, s):
    rl = (r0, r1)
    pltpu.sync_copy(idx_hbm.at[wid, 0], il[0])
    pltpu.async_copy(x_hbm.at[il[0].at[0]], r0, gsem)
    pltpu.sync_copy(idx_hbm.at[wid, 1], il[1])
    pltpu.async_copy(idx_hbm.at[wid, 2], il[2], isem)
    pltpu.async_copy(r1, acc.at[il[0].at[1]], ssem, add=True)

    def quad(t, _):
        j = t * 4
        for b in range(4):
            pltpu.make_async_copy(rl[(b + 1) % 2], acc.at[il[0].at[1]],
                                  ssem).wait()
            pltpu.async_copy(x_hbm.at[il[(b + 1) % 4].at[0]],
                             rl[(b + 1) % 2], gsem)
            pltpu.async_copy(idx_hbm.at[wid, j + b + 3], il[(b + 3) % 4],
                             isem)
            pltpu.make_async_copy(x_hbm.at[il[0].at[0]], rl[b % 2],
                                  gsem).wait()
            pltpu.make_async_copy(idx_hbm.at[wid, 0], il[(b + 2) % 4],
                                  isem).wait()
            pltpu.async_copy(rl[b % 2], acc.at[il[b % 4].at[1]], ssem,
                             add=True)
        return 0

    lax.fori_loop(0, _NCHUNK // 4, quad, 0)

    pltpu.make_async_copy(x_hbm.at[il[0].at[0]], r0, gsem).wait()
    pltpu.make_async_copy(idx_hbm.at[wid, 0], il[0], isem).wait()
    pltpu.make_async_copy(r0, acc.at[il[0].at[1]], ssem).wait()
    plsc.subcore_barrier()
    _write_out(acc, out_hbm, c, s)


def _sc_agg_body(x_hbm, idx_hbm, out_hbm, i0, i1, i2, i3, r0, r1, acc,
                 isem, gsem, ssem):
    c = lax.axis_index("c")
    s = lax.axis_index("s")
    wid = s * _NC + c

    _fill_vmem(r1, _CH, _D, 0.0)
    _zero_acc(r1, acc, s)
    plsc.subcore_barrier()

    _agg_phase(x_hbm, idx_hbm, out_hbm, (i0, i1, i2, i3), r0, r1, acc,
               isem, gsem, ssem, wid, c, s)


_sc_agg = pl.kernel(
    _sc_agg_body,
    out_type=jax.ShapeDtypeStruct((_NC, _NP, _D), jnp.float32),
    mesh=plsc.VectorSubcoreMesh(core_axis_name="c", subcore_axis_name="s"),
    scratch_types=(
        [pltpu.VMEM((2, _CH), jnp.int32) for _ in range(4)]
        + [pltpu.VMEM((_CH, _D), jnp.float32) for _ in range(2)]
        + [pltpu.VMEM_SHARED((_NP, _D), jnp.float32),
           pltpu.SemaphoreType.DMA,
           pltpu.SemaphoreType.DMA,
           pltpu.SemaphoreType.DMA]
    ),
)


def _sc_cnt_body(dst_hbm, out_hbm, hist, dst_all, tmp, cbuf, shared):
    c = lax.axis_index("c")
    s = lax.axis_index("s")
    wid = s * _NC + c

    _fill_vmem1d(hist, _NP, 0.0)
    pltpu.sync_copy(dst_hbm.at[wid], dst_all)

    def hrow(j, _):
        def hcol(l, __):
            v = dst_all[j, pl.ds(l * 16, 16)]
            cc, last = plsc.scan_count(v)
            plsc.addupdate_scatter(hist, [v], cc.astype(jnp.float32),
                                   mask=last)
            return 0
        return lax.fori_loop(0, _CH // 16, hcol, 0)

    lax.fori_loop(0, _NCHUNK, hrow, 0)

    pltpu.sync_copy(hist, shared.at[s])
    plsc.subcore_barrier()

    _fill_vmem1d(cbuf, _RPT, 0.0)
    for t in range(_NS):
        pltpu.sync_copy(shared.at[t, pl.ds(s * _RPT, _RPT)], tmp)

        def add(v, _):
            cbuf[pl.ds(v * 16, 16)] = cbuf[pl.ds(v * 16, 16)] + tmp[pl.ds(v * 16, 16)]
            return 0
        lax.fori_loop(0, _RPT // 16, add, 0)

    pltpu.sync_copy(cbuf, out_hbm.at[c, s])


_sc_cnt = pl.kernel(
    _sc_cnt_body,
    out_type=jax.ShapeDtypeStruct((_NC, _NS, _RPT), jnp.float32),
    mesh=plsc.VectorSubcoreMesh(core_axis_name="c", subcore_axis_name="s"),
    scratch_types=[
        pltpu.VMEM((_NP,), jnp.float32),
        pltpu.VMEM((_NCHUNK, _CH), jnp.int32),
        pltpu.VMEM((_RPT,), jnp.float32),
        pltpu.VMEM((_RPT,), jnp.float32),
        pltpu.VMEM_SHARED((_NS, _NP), jnp.float32),
    ],
    compiler_params=pltpu.CompilerParams(needs_layout_passes=False),
)


def _pad_edges(edge_index):
    src = edge_index[0].reshape(_NW, _E // _NW)
    dst = edge_index[1].reshape(_NW, _E // _NW)
    pad_src = (jnp.arange(_NW * _NPAD, dtype=jnp.int32) % _N).reshape(_NW, _NPAD)
    pad_dst = (_N + jnp.arange(_NW * _NPAD, dtype=jnp.int32) % (_NP - _N)
               ).reshape(_NW, _NPAD)
    src = jnp.concatenate([src, pad_src], axis=1).reshape(_NW, _NCHUNK, _CH)
    dst = jnp.concatenate([dst, pad_dst], axis=1).reshape(_NW, _NCHUNK, _CH)
    idx4 = jnp.stack([src, dst], axis=2)
    idx4 = jnp.concatenate([idx4, idx4[:, :4]], axis=1)
    return idx4, dst


_RB = 2000


def _tc_layer1_body(x_ref, p_ref, cnt_ref, wl_ref, bl_ref, wr_ref, h_ref):
    cnt = cnt_ref[0] + cnt_ref[1]
    mean = (p_ref[0] + p_ref[1]) / jnp.maximum(cnt, 1.0)
    dn = (((1,), (1,)), ((), ()))
    h = (lax.dot_general(x_ref[...], wl_ref[...], dn,
                         preferred_element_type=jnp.float32)
         + bl_ref[...]
         + lax.dot_general(mean, wr_ref[...], dn,
                           preferred_element_type=jnp.float32))
    h_ref[...] = jnp.maximum(h, 0.0)


def _tc_layer2_body(h_ref, p_ref, cnt_ref, wl_ref, bl_ref, wr_ref,
                    wo_ref, bo_ref, out_ref):
    cnt = cnt_ref[0] + cnt_ref[1]
    mean = (p_ref[0] + p_ref[1]) / jnp.maximum(cnt, 1.0)
    dn = (((1,), (1,)), ((), ()))
    h2 = (lax.dot_general(h_ref[...], wl_ref[...], dn,
                          preferred_element_type=jnp.float32)
          + bl_ref[...]
          + lax.dot_general(mean, wr_ref[...], dn,
                            preferred_element_type=jnp.float32))
    h2 = jnp.maximum(h2, 0.0)
    out_ref[...] = lax.dot_general(h2, wo_ref[...], dn,
                                   preferred_element_type=jnp.float32) + bo_ref[...]


def _row_spec():
    return pl.BlockSpec((_RB, _D), lambda i: (i, 0))


def _part_spec():
    return pl.BlockSpec((_NC, _RB, _D), lambda i: (0, i, 0))


def _cnt_spec():
    return pl.BlockSpec((_NC, _RB, 1), lambda i: (0, i, 0))


def _w_spec():
    return pl.BlockSpec((_D, _D), lambda i: (0, 0))


def _b_spec():
    return pl.BlockSpec((_D,), lambda i: (0,))


def _tc_layer1(x, p, cntp, Wl, bl, Wr):
    return pl.pallas_call(
        _tc_layer1_body,
        grid=(_N // _RB,),
        in_specs=[_row_spec(), _part_spec(), _cnt_spec(),
                  _w_spec(), _b_spec(), _w_spec()],
        out_specs=_row_spec(),
        out_shape=jax.ShapeDtypeStruct((_N, _D), jnp.float32),
    )(x, p, cntp, Wl, bl, Wr)


def _tc_layer2(h, p, cntp, Wl, bl, Wr, Wo, bo):
    return pl.pallas_call(
        _tc_layer2_body,
        grid=(_N // _RB,),
        in_specs=[_row_spec(), _part_spec(), _cnt_spec(),
                  _w_spec(), _b_spec(), _w_spec(), _w_spec(), _b_spec()],
        out_specs=_row_spec(),
        out_shape=jax.ShapeDtypeStruct((_N, _D), jnp.float32),
    )(h, p, cntp, Wl, bl, Wr, Wo, bo)


def kernel(x, edge_index, W1l, b1l, W1r, W2l, b2l, W2r, Wout, bout):
    idx4, dst3 = _pad_edges(edge_index)
    cntp = _sc_cnt(dst3).reshape(_NC, _NP, 1)
    p1 = _sc_agg(x, idx4)
    h = _tc_layer1(x, p1, cntp, W1l, b1l, W1r)
    p2 = _sc_agg(h, idx4)
    return _tc_layer2(h, p2, cntp, W2l, b2l, W2r, Wout, bout)


# speedup vs baseline: 1.6987x; 1.0004x over previous
import jax
import jax.numpy as jnp
from jax import lax
from jax.experimental import pallas as pl
from jax.experimental.pallas import tpu as pltpu
from jax.experimental.pallas import tpu_sc as plsc

_N = 10000
_E = 320000
_D = 128

_NC = 2
_NS = 16
_NW = _NC * _NS
_CH = 128
_NCHUNK = 80
_EPWP = _CH * _NCHUNK
_NPAD = _EPWP - _E // _NW
_NP = 10240
_RPT = _NP // _NS


def _fill_vmem1d(ref, n, value):
    v = jnp.full((16,), value, jnp.float32)

    def step(i, _):
        ref[pl.ds(i * 16, 16)] = v
        return 0

    lax.fori_loop(0, n // 16, step, 0)


def _fill_vmem(ref, rows, cols, value):
    v = jnp.full((16,), value, jnp.float32)

    def row(i, _):
        def col(j, __):
            ref[i, pl.ds(j * 16, 16)] = v
            return 0
        return lax.fori_loop(0, cols // 16, col, 0)

    lax.fori_loop(0, rows, row, 0)


def _zero_acc(zbuf, acc, s):
    for k in range(_RPT // _CH):
        pltpu.sync_copy(zbuf, acc.at[pl.ds(s * _RPT + k * _CH, _CH)])


def _write_out(acc, out_hbm, c, s):
    pltpu.sync_copy(acc.at[pl.ds(s * _RPT, _RPT)],
                    out_hbm.at[c, pl.ds(s * _RPT, _RPT)])


def _agg_phase(x_hbm, idx_hbm, out_hbm, il, r0, r1, acc, isem, gsem, ssem,
               wid, c, s):
    rl = (r0, r1)
    pltpu.sync_copy(idx_hbm.at[wid, 0], il[0])
    pltpu.async_copy(x_hbm.at[il[0].at[0]], r0, gsem)
    pltpu.sync_copy(idx_hbm.at[wid, 1], il[1])
    pltpu.async_copy(idx_hbm.at[wid, 2], il[2], isem)
    pltpu.async_copy(r1, acc.at[il[0].at[1]], ssem, add=True)

    def quad(t, _):
        j = t * 4
        for b in range(4):
            pltpu.make_async_copy(rl[(b + 1) % 2], acc.at[il[0].at[1]],
                                  ssem).wait()
            pltpu.async_copy(x_hbm.at[il[(b + 1) % 4].at[0]],
                             rl[(b + 1) % 2], gsem)
            pltpu.async_copy(idx_hbm.at[wid, j + b + 3], il[(b + 3) % 4],
                             isem)
            pltpu.make_async_copy(x_hbm.at[il[0].at[0]], rl[b % 2],
                                  gsem).wait()
            pltpu.make_async_copy(idx_hbm.at[wid, 0], il[(b + 2) % 4],
                                  isem).wait()
            pltpu.async_copy(rl[b % 2], acc.at[il[b % 4].at[1]], ssem,
                             add=True)
        return 0

    lax.fori_loop(0, _NCHUNK // 4, quad, 0)

    pltpu.make_async_copy(x_hbm.at[il[0].at[0]], r0, gsem).wait()
    pltpu.make_async_copy(idx_hbm.at[wid, 0], il[0], isem).wait()
    pltpu.make_async_copy(r0, acc.at[il[0].at[1]], ssem).wait()
    plsc.subcore_barrier()
    _write_out(acc, out_hbm, c, s)


def _sc_agg_body(x_hbm, idx_hbm, out_hbm, i0, i1, i2, i3, r0, r1, acc,
                 isem, gsem, ssem):
    c = lax.axis_index("c")
    s = lax.axis_index("s")
    wid = s * _NC + c

    _fill_vmem(r1, _CH, _D, 0.0)
    _zero_acc(r1, acc, s)
    plsc.subcore_barrier()

    _agg_phase(x_hbm, idx_hbm, out_hbm, (i0, i1, i2, i3), r0, r1, acc,
               isem, gsem, ssem, wid, c, s)


def _make_sc_agg():
    return pl.kernel(
    _sc_agg_body,
    out_type=jax.ShapeDtypeStruct((_NC, _NP, _D), jnp.float32),
    mesh=plsc.VectorSubcoreMesh(core_axis_name="c", subcore_axis_name="s"),
    scratch_types=(
        [pltpu.VMEM((2, _CH), jnp.int32) for _ in range(4)]
        + [pltpu.VMEM((_CH, _D), jnp.float32) for _ in range(2)]
        + [pltpu.VMEM_SHARED((_NP, _D), jnp.float32),
           pltpu.SemaphoreType.DMA,
           pltpu.SemaphoreType.DMA,
           pltpu.SemaphoreType.DMA]
    ),
    )


def _sc_cnt_body(dst_hbm, out_hbm, hist, dst_all, tmp, cbuf, shared):
    c = lax.axis_index("c")
    s = lax.axis_index("s")
    wid = s * _NC + c

    _fill_vmem1d(hist, _NP, 0.0)
    pltpu.sync_copy(dst_hbm.at[wid], dst_all)

    def hrow(j, _):
        def hcol(l, __):
            v = dst_all[j, pl.ds(l * 16, 16)]
            cc, last = plsc.scan_count(v)
            plsc.addupdate_scatter(hist, [v], cc.astype(jnp.float32),
                                   mask=last)
            return 0
        return lax.fori_loop(0, _CH // 16, hcol, 0)

    lax.fori_loop(0, _NCHUNK, hrow, 0)

    pltpu.sync_copy(hist, shared.at[s])
    plsc.subcore_barrier()

    _fill_vmem1d(cbuf, _RPT, 0.0)
    for t in range(_NS):
        pltpu.sync_copy(shared.at[t, pl.ds(s * _RPT, _RPT)], tmp)

        def add(v, _):
            cbuf[pl.ds(v * 16, 16)] = cbuf[pl.ds(v * 16, 16)] + tmp[pl.ds(v * 16, 16)]
            return 0
        lax.fori_loop(0, _RPT // 16, add, 0)

    pltpu.sync_copy(cbuf, out_hbm.at[c, s])


def _make_sc_cnt():
    return pl.kernel(
    _sc_cnt_body,
    out_type=jax.ShapeDtypeStruct((_NC, _NS, _RPT), jnp.float32),
    mesh=plsc.VectorSubcoreMesh(core_axis_name="c", subcore_axis_name="s"),
    scratch_types=[
        pltpu.VMEM((_NP,), jnp.float32),
        pltpu.VMEM((_NCHUNK, _CH), jnp.int32),
        pltpu.VMEM((_RPT,), jnp.float32),
        pltpu.VMEM((_RPT,), jnp.float32),
        pltpu.VMEM_SHARED((_NS, _NP), jnp.float32),
    ],
    compiler_params=pltpu.CompilerParams(needs_layout_passes=False),
    )


_sc_kernels = {}


def _get_sc_kernels():
    if "agg" not in _sc_kernels:
        _sc_kernels["agg"] = _make_sc_agg()
        _sc_kernels["cnt"] = _make_sc_cnt()
    return _sc_kernels["agg"], _sc_kernels["cnt"]


def _pad_edges(edge_index):
    src = edge_index[0].reshape(_NW, _E // _NW)
    dst = edge_index[1].reshape(_NW, _E // _NW)
    pad_src = (jnp.arange(_NW * _NPAD, dtype=jnp.int32) % _N).reshape(_NW, _NPAD)
    pad_dst = (_N + jnp.arange(_NW * _NPAD, dtype=jnp.int32) % (_NP - _N)
               ).reshape(_NW, _NPAD)
    src = jnp.concatenate([src, pad_src], axis=1).reshape(_NW, _NCHUNK, _CH)
    dst = jnp.concatenate([dst, pad_dst], axis=1).reshape(_NW, _NCHUNK, _CH)
    idx4 = jnp.stack([src, dst], axis=2)
    idx4 = jnp.concatenate([idx4, idx4[:, :4]], axis=1)
    return idx4, dst


_RB = 2000


def _tc_layer1_body(x_ref, p_ref, cnt_ref, wl_ref, bl_ref, wr_ref, h_ref):
    cnt = cnt_ref[0] + cnt_ref[1]
    mean = (p_ref[0] + p_ref[1]) / jnp.maximum(cnt, 1.0)
    dn = (((1,), (1,)), ((), ()))
    h = (lax.dot_general(x_ref[...], wl_ref[...], dn,
                         preferred_element_type=jnp.float32)
         + bl_ref[...]
         + lax.dot_general(mean, wr_ref[...], dn,
                           preferred_element_type=jnp.float32))
    h_ref[...] = jnp.maximum(h, 0.0)


def _tc_layer2_body(h_ref, p_ref, cnt_ref, wl_ref, bl_ref, wr_ref,
                    wo_ref, bo_ref, out_ref):
    cnt = cnt_ref[0] + cnt_ref[1]
    mean = (p_ref[0] + p_ref[1]) / jnp.maximum(cnt, 1.0)
    dn = (((1,), (1,)), ((), ()))
    h2 = (lax.dot_general(h_ref[...], wl_ref[...], dn,
                          preferred_element_type=jnp.float32)
          + bl_ref[...]
          + lax.dot_general(mean, wr_ref[...], dn,
                            preferred_element_type=jnp.float32))
    h2 = jnp.maximum(h2, 0.0)
    out_ref[...] = lax.dot_general(h2, wo_ref[...], dn,
                                   preferred_element_type=jnp.float32) + bo_ref[...]


def _row_spec():
    return pl.BlockSpec((_RB, _D), lambda i: (i, 0))


def _part_spec():
    return pl.BlockSpec((_NC, _RB, _D), lambda i: (0, i, 0))


def _cnt_spec():
    return pl.BlockSpec((_NC, _RB, 1), lambda i: (0, i, 0))


def _w_spec():
    return pl.BlockSpec((_D, _D), lambda i: (0, 0))


def _b_spec():
    return pl.BlockSpec((_D,), lambda i: (0,))


def _tc_layer1(x, p, cntp, Wl, bl, Wr):
    return pl.pallas_call(
        _tc_layer1_body,
        grid=(_N // _RB,),
        in_specs=[_row_spec(), _part_spec(), _cnt_spec(),
                  _w_spec(), _b_spec(), _w_spec()],
        out_specs=_row_spec(),
        out_shape=jax.ShapeDtypeStruct((_N, _D), jnp.float32),
    )(x, p, cntp, Wl, bl, Wr)


def _tc_layer2(h, p, cntp, Wl, bl, Wr, Wo, bo):
    return pl.pallas_call(
        _tc_layer2_body,
        grid=(_N // _RB,),
        in_specs=[_row_spec(), _part_spec(), _cnt_spec(),
                  _w_spec(), _b_spec(), _w_spec(), _w_spec(), _b_spec()],
        out_specs=_row_spec(),
        out_shape=jax.ShapeDtypeStruct((_N, _D), jnp.float32),
    )(h, p, cntp, Wl, bl, Wr, Wo, bo)


def kernel(x, edge_index, W1l, b1l, W1r, W2l, b2l, W2r, Wout, bout):
    _sc_agg, _sc_cnt = _get_sc_kernels()
    idx4, dst3 = _pad_edges(edge_index)
    cntp = _sc_cnt(dst3).reshape(_NC, _NP, 1)
    p1 = _sc_agg(x, idx4)
    h = _tc_layer1(x, p1, cntp, W1l, b1l, W1r)
    p2 = _sc_agg(h, idx4)
    return _tc_layer2(h, p2, cntp, W2l, b2l, W2r, Wout, bout)
